# baseline (speedup 1.0000x reference)
import functools

import jax
import jax.numpy as jnp
from jax import lax
from jax.experimental import pallas as pl
from jax.experimental.pallas import tpu as pltpu
from jax.experimental.pallas import tpu_sc as plsc

B, S, D, V = 4, 2048, 768, 100000
NC, NS = 2, 16
NW = NC * NS
CHUNK = S // NW
W = 32
NHALF = CHUNK // W
NSUB = B * NHALF
NBUF = 3
LANES = 16


def _build():
    mesh = plsc.VectorSubcoreMesh(core_axis_name="c", subcore_axis_name="s")

    @functools.partial(
        pl.kernel,
        mesh=mesh,
        out_type=jax.ShapeDtypeStruct((B, S, D), jnp.float32),
        scratch_types=[
            pltpu.VMEM((B, CHUNK), jnp.int32),
            pltpu.VMEM((CHUNK, D), jnp.float32),
            pltpu.VMEM((W, D), jnp.float32),
            pltpu.VMEM((W, D), jnp.float32),
            pltpu.VMEM((W, D), jnp.float32),
            pltpu.SemaphoreType.DMA,
            pltpu.SemaphoreType.DMA,
            pltpu.SemaphoreType.DMA,
        ],
    )
    def emb_kernel(x_hbm, tok_hbm, pos_hbm, out_hbm,
                   idx_v, pos_v, buf0, buf1, buf2, gsem, ssem, psem):
        wid = lax.axis_index("s") * NC + lax.axis_index("c")
        base = wid * CHUNK
        bufs = (buf0, buf1, buf2)

        pos_cp = pltpu.async_copy(pos_hbm.at[pl.ds(base, CHUNK)], pos_v, psem)
        idx_cps = [
            pltpu.async_copy(x_hbm.at[b, pl.ds(base, CHUNK)], idx_v.at[b], gsem)
            for b in range(B)
        ]
        for cp in idx_cps:
            cp.wait()

        def gather(k):
            b, h = divmod(k, NHALF)
            return pltpu.async_copy(
                tok_hbm.at[idx_v.at[b, pl.ds(h * W, W)]], bufs[k % NBUF], gsem)

        def store(k):
            b, h = divmod(k, NHALF)
            return pltpu.async_copy(
                bufs[k % NBUF], out_hbm.at[b, pl.ds(base + h * W, W)], ssem)

        gathers = [None] * NSUB
        stores = [None] * NSUB
        gathers[0] = gather(0)
        gathers[1] = gather(1)
        for k in range(NSUB):
            if k + 2 < NSUB:
                if k + 2 >= NBUF:
                    stores[k + 2 - NBUF].wait()
                gathers[k + 2] = gather(k + 2)
            gathers[k].wait()
            if k == 0:
                pos_cp.wait()

            h = k % NHALF
            buf = bufs[k % NBUF]

            def add_row(r, _):
                for j in range(D // LANES):
                    sl = pl.ds(j * LANES, LANES)
                    plsc.addupdate(buf.at[r, sl], pos_v[h * W + r, sl])
                return 0

            lax.fori_loop(0, W, add_row, 0)
            stores[k] = store(k)
        for k in range(NSUB - NBUF, NSUB):
            stores[k].wait()

    return emb_kernel


_emb = _build()


def kernel(x, token_emb, pos_emb):
    return _emb(x.astype(jnp.int32), token_emb, pos_emb)


# speedup vs baseline: 1.0453x; 1.0453x over previous
import functools

import jax
import jax.numpy as jnp
from jax import lax
from jax.experimental import pallas as pl
from jax.experimental.pallas import tpu as pltpu
from jax.experimental.pallas import tpu_sc as plsc

B, S, D, V = 4, 2048, 768, 100000
NC, NS = 2, 16
NW = NC * NS
CHUNK = S // NW
W = 32
NHALF = CHUNK // W
NSUB = B * NHALF
NBUF = 3
LANES = 16


def _build():
    mesh = plsc.VectorSubcoreMesh(core_axis_name="c", subcore_axis_name="s")

    @functools.partial(
        pl.kernel,
        mesh=mesh,
        out_type=jax.ShapeDtypeStruct((B, S, D), jnp.float32),
        scratch_types=[
            pltpu.VMEM((NSUB, W), jnp.int32),
            pltpu.VMEM((CHUNK, D), jnp.float32),
            pltpu.VMEM((W, D), jnp.float32),
            pltpu.VMEM((W, D), jnp.float32),
            pltpu.VMEM((W, D), jnp.float32),
            pltpu.SemaphoreType.DMA,
            pltpu.SemaphoreType.DMA,
            pltpu.SemaphoreType.DMA,
        ],
    )
    def emb_kernel(x_hbm, tok_hbm, pos_hbm, out_hbm,
                   idx_v, pos_v, buf0, buf1, buf2, gsem, ssem, psem):
        wid = lax.axis_index("s") * NC + lax.axis_index("c")
        base = wid * CHUNK
        bufs = (buf0, buf1, buf2)

        pos_cp = pltpu.async_copy(pos_hbm.at[pl.ds(base, CHUNK)], pos_v, psem)
        idx_cps = [
            pltpu.async_copy(
                x_hbm.at[k // NHALF, pl.ds(base + (k % NHALF) * W, W)],
                idx_v.at[k], gsem)
            for k in range(NSUB)
        ]
        for cp in idx_cps:
            cp.wait()

        def gather(k):
            return pltpu.async_copy(
                tok_hbm.at[idx_v.at[k]], bufs[k % NBUF], gsem)

        def store(k):
            b, h = divmod(k, NHALF)
            return pltpu.async_copy(
                bufs[k % NBUF], out_hbm.at[b, pl.ds(base + h * W, W)], ssem)

        gathers = [None] * NSUB
        stores = [None] * NSUB
        gathers[0] = gather(0)
        for k in range(NSUB):
            if k + 1 < NSUB:
                if k + 1 >= NBUF:
                    stores[k + 1 - NBUF].wait()
                gathers[k + 1] = gather(k + 1)
            gathers[k].wait()
            if k == 0:
                pos_cp.wait()

            h = k % NHALF
            buf = bufs[k % NBUF]

            def add_row(r, _):
                for j in range(D // LANES):
                    sl = pl.ds(j * LANES, LANES)
                    plsc.addupdate(buf.at[r, sl], pos_v[h * W + r, sl])
                return 0

            lax.fori_loop(0, W, add_row, 0)
            stores[k] = store(k)
        for k in range(NSUB - NBUF, NSUB):
            stores[k].wait()

    return emb_kernel


_emb = _build()


def kernel(x, token_emb, pos_emb):
    return _emb(x.astype(jnp.int32), token_emb, pos_emb)


# speedup vs baseline: 1.2018x; 1.1497x over previous
import functools

import jax
import jax.numpy as jnp
from jax import lax
from jax.experimental import pallas as pl
from jax.experimental.pallas import tpu as pltpu
from jax.experimental.pallas import tpu_sc as plsc

B, S, D, V = 4, 2048, 768, 100000
NC, NS = 2, 16
NW = NC * NS
CHUNK = S // NW
W = 16
NQ = CHUNK // W
NG = 2
LANES = 16
GRP = D // LANES


def _build():
    mesh = plsc.VectorSubcoreMesh(core_axis_name="c", subcore_axis_name="s")

    tok_scratch = [pltpu.VMEM((W, D), jnp.float32) for _ in range(NG * B)]
    pos_scratch = [pltpu.VMEM((W, D), jnp.float32) for _ in range(NG)]

    @functools.partial(
        pl.kernel,
        mesh=mesh,
        out_type=jax.ShapeDtypeStruct((B, S, D), jnp.float32),
        scratch_types=[
            pltpu.VMEM((B, CHUNK), jnp.int32),
            *tok_scratch,
            *pos_scratch,
            pltpu.SemaphoreType.DMA,
            pltpu.SemaphoreType.DMA,
            pltpu.SemaphoreType.DMA,
        ],
    )
    def emb_kernel(x_hbm, tok_hbm, pos_hbm, out_hbm, idx_v, *rest):
        tok_bufs = rest[:NG * B]
        pos_bufs = rest[NG * B:NG * B + NG]
        fsem, gsem, ssem = rest[NG * B + NG:]

        wid = lax.axis_index("s") * NC + lax.axis_index("c")
        base = wid * CHUNK

        idx_cps = [
            pltpu.async_copy(x_hbm.at[b, pl.ds(base, CHUNK)], idx_v.at[b], gsem)
            for b in range(B)
        ]
        for cp in idx_cps:
            cp.wait()

        def fill(q):
            return pltpu.async_copy(
                pos_hbm.at[pl.ds(base + q * W, W)], pos_bufs[q % NG], fsem)

        def gather(q, b):
            return pltpu.async_copy(
                tok_hbm.at[idx_v.at[b, pl.ds(q * W, W)]],
                tok_bufs[(q % NG) * B + b], gsem)

        def store(q, b):
            return pltpu.async_copy(
                tok_bufs[(q % NG) * B + b],
                out_hbm.at[b, pl.ds(base + q * W, W)], ssem)

        fills = [None] * NQ
        gs = [None] * NQ
        ss = [None] * NQ
        fills[0] = fill(0)
        gs[0] = [gather(0, b) for b in range(B)]
        for q in range(NQ):
            if q + 1 < NQ:
                if q + 1 >= NG:
                    for h in ss[q + 1 - NG]:
                        h.wait()
                fills[q + 1] = fill(q + 1)
                gs[q + 1] = [gather(q + 1, b) for b in range(B)]
            fills[q].wait()
            for h in gs[q]:
                h.wait()

            grp_bufs = tok_bufs[(q % NG) * B:(q % NG) * B + B]
            pos_b = pos_bufs[q % NG]

            def add_row(r, _):
                pvec = [pos_b[r, pl.ds(j * LANES, LANES)] for j in range(GRP)]
                for b in range(B):
                    buf = grp_bufs[b]
                    for j in range(GRP):
                        plsc.addupdate(buf.at[r, pl.ds(j * LANES, LANES)], pvec[j])
                return 0

            lax.fori_loop(0, W, add_row, 0)
            ss[q] = [store(q, b) for b in range(B)]
        for q in range(NQ - NG, NQ):
            for h in ss[q]:
                h.wait()

    return emb_kernel


_emb = _build()


def kernel(x, token_emb, pos_emb):
    return _emb(x.astype(jnp.int32), token_emb, pos_emb)


# speedup vs baseline: 1.2521x; 1.0419x over previous
import functools

import jax
import jax.numpy as jnp
from jax import lax
from jax.experimental import pallas as pl
from jax.experimental.pallas import tpu as pltpu
from jax.experimental.pallas import tpu_sc as plsc

B, S, D, V = 4, 2048, 768, 100000
NC, NS = 2, 16
NW = NC * NS
CHUNK = S // NW
W = 16
NQ = CHUNK // W
NG = 2
LANES = 16
GRP = D // LANES


def _build():
    mesh = plsc.VectorSubcoreMesh(core_axis_name="c", subcore_axis_name="s")

    tok_scratch = [pltpu.VMEM((W, D), jnp.float32) for _ in range(NG * B)]
    pos_scratch = [pltpu.VMEM((W, D), jnp.float32) for _ in range(NG)]

    @functools.partial(
        pl.kernel,
        mesh=mesh,
        out_type=jax.ShapeDtypeStruct((B, S, D), jnp.float32),
        scratch_types=[
            pltpu.VMEM((B, CHUNK), jnp.int32),
            *tok_scratch,
            *pos_scratch,
            pltpu.SemaphoreType.DMA,
            pltpu.SemaphoreType.DMA,
            pltpu.SemaphoreType.DMA,
        ],
    )
    def emb_kernel(x_hbm, tok_hbm, pos_hbm, out_hbm, idx_v, *rest):
        tok_bufs = rest[:NG * B]
        pos_bufs = rest[NG * B:NG * B + NG]
        fsem, gsem, ssem = rest[NG * B + NG:]

        wid = lax.axis_index("s") * NC + lax.axis_index("c")
        base = wid * CHUNK

        idx_cps = [
            pltpu.async_copy(x_hbm.at[b, pl.ds(base, CHUNK)], idx_v.at[b], gsem)
            for b in range(B)
        ]
        for cp in idx_cps:
            cp.wait()

        def fill(q):
            return pltpu.async_copy(
                pos_hbm.at[pl.ds(base + q * W, W)], pos_bufs[q % NG], fsem)

        def gather(q, b):
            return pltpu.async_copy(
                tok_hbm.at[idx_v.at[b, pl.ds(q * W, W)]],
                tok_bufs[(q % NG) * B + b], gsem)

        def store(q, b):
            return pltpu.async_copy(
                tok_bufs[(q % NG) * B + b],
                out_hbm.at[b, pl.ds(base + q * W, W)], ssem)

        fills = [None] * NQ
        gs = [None] * NQ
        ss = [None] * NQ
        fills[0] = fill(0)
        gs[0] = [gather(0, b) for b in range(B)]
        for q in range(NQ):
            if q + 1 < NQ:
                if q + 1 >= NG:
                    for h in ss[q + 1 - NG]:
                        h.wait()
                fills[q + 1] = fill(q + 1)
                gs[q + 1] = [gather(q + 1, b) for b in range(B)]
            fills[q].wait()
            for h in gs[q]:
                h.wait()

            grp_bufs = tok_bufs[(q % NG) * B:(q % NG) * B + B]
            pos_b = pos_bufs[q % NG]

            def add_row(r, _):
                for half in range(2):
                    j0 = half * (GRP // 2)
                    pvec = [pos_b[r, pl.ds((j0 + j) * LANES, LANES)]
                            for j in range(GRP // 2)]
                    for b in range(B):
                        buf = grp_bufs[b]
                        for j in range(GRP // 2):
                            plsc.addupdate(
                                buf.at[r, pl.ds((j0 + j) * LANES, LANES)], pvec[j])
                return 0

            lax.fori_loop(0, W, add_row, 0)
            ss[q] = [store(q, b) for b in range(B)]
        for q in range(NQ - NG, NQ):
            for h in ss[q]:
                h.wait()

    return emb_kernel


_emb = _build()


def kernel(x, token_emb, pos_emb):
    return _emb(x.astype(jnp.int32), token_emb, pos_emb)


# speedup vs baseline: 1.2774x; 1.0202x over previous
import functools

import jax
import jax.numpy as jnp
from jax import lax
from jax.experimental import pallas as pl
from jax.experimental.pallas import tpu as pltpu
from jax.experimental.pallas import tpu_sc as plsc

B, S, D, V = 4, 2048, 768, 100000
NC, NS = 2, 16
NW = NC * NS
CHUNK = S // NW
W = 16
NQ = CHUNK // W
NG = 2
LANES = 16
GRP = D // LANES


def _build():
    mesh = plsc.VectorSubcoreMesh(core_axis_name="c", subcore_axis_name="s")

    tok_scratch = [pltpu.VMEM((W, D), jnp.float32) for _ in range(NG * B)]
    pos_scratch = [pltpu.VMEM((W, D), jnp.float32) for _ in range(NG)]

    @functools.partial(
        pl.kernel,
        mesh=mesh,
        out_type=jax.ShapeDtypeStruct((B, S, D), jnp.float32),
        scratch_types=[
            pltpu.VMEM((B, CHUNK), jnp.int32),
            *tok_scratch,
            *pos_scratch,
            pltpu.SemaphoreType.DMA,
            pltpu.SemaphoreType.DMA,
            pltpu.SemaphoreType.DMA,
        ],
    )
    def emb_kernel(x_hbm, tok_hbm, pos_hbm, out_hbm, idx_v, *rest):
        tok_bufs = rest[:NG * B]
        pos_bufs = rest[NG * B:NG * B + NG]
        fsem, gsem, ssem = rest[NG * B + NG:]

        wid = lax.axis_index("s") * NC + lax.axis_index("c")
        base = wid * CHUNK

        idx_cps = [
            pltpu.async_copy(x_hbm.at[b, pl.ds(base, CHUNK)], idx_v.at[b], gsem)
            for b in range(B)
        ]
        for cp in idx_cps:
            cp.wait()

        def fill(q):
            return pltpu.async_copy(
                pos_hbm.at[pl.ds(base + q * W, W)], pos_bufs[q % NG], fsem)

        def gather(q, b):
            return pltpu.async_copy(
                tok_hbm.at[idx_v.at[b, pl.ds(q * W, W)]],
                tok_bufs[(q % NG) * B + b], gsem)

        def store(q, b):
            return pltpu.async_copy(
                tok_bufs[(q % NG) * B + b],
                out_hbm.at[b, pl.ds(base + q * W, W)], ssem)

        fills = [None] * NQ
        gs = [None] * NQ
        ss = [None] * NQ
        fills[0] = fill(0)
        gs[0] = [gather(0, b) for b in range(B)]
        for q in range(NQ):
            if q + 1 < NQ:
                if q + 1 >= NG:
                    for h in ss[q + 1 - NG]:
                        h.wait()
                fills[q + 1] = fill(q + 1)
                gs[q + 1] = [gather(q + 1, b) for b in range(B)]
            fills[q].wait()
            for h in gs[q]:
                h.wait()

            grp_bufs = tok_bufs[(q % NG) * B:(q % NG) * B + B]
            pos_b = pos_bufs[q % NG]

            def add_row(r, _):
                for half in range(4):
                    j0 = half * (GRP // 4)
                    pvec = [pos_b[r, pl.ds((j0 + j) * LANES, LANES)]
                            for j in range(GRP // 4)]
                    for b in range(B):
                        buf = grp_bufs[b]
                        for j in range(GRP // 4):
                            plsc.addupdate(
                                buf.at[r, pl.ds((j0 + j) * LANES, LANES)], pvec[j])
                return 0

            lax.fori_loop(0, W, add_row, 0)
            ss[q] = [store(q, b) for b in range(B)]
        for q in range(NQ - NG, NQ):
            for h in ss[q]:
                h.wait()

    return emb_kernel


_emb = _build()


def kernel(x, token_emb, pos_emb):
    return _emb(x.astype(jnp.int32), token_emb, pos_emb)


# speedup vs baseline: 1.2911x; 1.0107x over previous
import functools

import jax
import jax.numpy as jnp
from jax import lax
from jax.experimental import pallas as pl
from jax.experimental.pallas import tpu as pltpu
from jax.experimental.pallas import tpu_sc as plsc

B, S, D, V = 4, 2048, 768, 100000
NC, NS = 2, 16
NW = NC * NS
CHUNK = S // NW
W = 16
NQ = CHUNK // W
NG = 2
LANES = 16
GRP = D // LANES


def _build():
    mesh = plsc.VectorSubcoreMesh(core_axis_name="c", subcore_axis_name="s")

    tok_scratch = [pltpu.VMEM((W, D), jnp.float32) for _ in range(NG * B)]
    pos_scratch = [pltpu.VMEM((W, D), jnp.float32) for _ in range(NG)]

    @functools.partial(
        pl.kernel,
        mesh=mesh,
        out_type=jax.ShapeDtypeStruct((B, S, D), jnp.float32),
        scratch_types=[
            pltpu.VMEM((B, CHUNK), jnp.int32),
            *tok_scratch,
            *pos_scratch,
            pltpu.SemaphoreType.DMA,
            pltpu.SemaphoreType.DMA,
            pltpu.SemaphoreType.DMA,
        ],
    )
    def emb_kernel(x_hbm, tok_hbm, pos_hbm, out_hbm, idx_v, *rest):
        tok_bufs = rest[:NG * B]
        pos_bufs = rest[NG * B:NG * B + NG]
        fsem, gsem, ssem = rest[NG * B + NG:]

        wid = lax.axis_index("s") * NC + lax.axis_index("c")
        base = wid * CHUNK

        idx_cps = [
            pltpu.async_copy(x_hbm.at[b, pl.ds(base, CHUNK)], idx_v.at[b], gsem)
            for b in range(B)
        ]

        def fill(q):
            return pltpu.async_copy(
                pos_hbm.at[pl.ds(base + q * W, W)], pos_bufs[q % NG], fsem)

        def gather(q, b):
            return pltpu.async_copy(
                tok_hbm.at[idx_v.at[b, pl.ds(q * W, W)]],
                tok_bufs[(q % NG) * B + b], gsem)

        def store(q, b):
            return pltpu.async_copy(
                tok_bufs[(q % NG) * B + b],
                out_hbm.at[b, pl.ds(base + q * W, W)], ssem)

        fills = [None] * NQ
        gs = [None] * NQ
        ss = [None] * NQ
        fills[0] = fill(0)
        for cp in idx_cps:
            cp.wait()
        gs[0] = [gather(0, b) for b in range(B)]
        for q in range(NQ):
            if q + 1 < NQ:
                if q + 1 >= NG:
                    for h in ss[q + 1 - NG]:
                        h.wait()
                fills[q + 1] = fill(q + 1)
                gs[q + 1] = [gather(q + 1, b) for b in range(B)]
            fills[q].wait()
            for h in gs[q]:
                h.wait()

            grp_bufs = tok_bufs[(q % NG) * B:(q % NG) * B + B]
            pos_b = pos_bufs[q % NG]

            def add_row(r, _):
                for half in range(4):
                    j0 = half * (GRP // 4)
                    pvec = [pos_b[r, pl.ds((j0 + j) * LANES, LANES)]
                            for j in range(GRP // 4)]
                    for b in range(B):
                        buf = grp_bufs[b]
                        for j in range(GRP // 4):
                            plsc.addupdate(
                                buf.at[r, pl.ds((j0 + j) * LANES, LANES)], pvec[j])
                return 0

            lax.fori_loop(0, W, add_row, 0)
            ss[q] = [store(q, b) for b in range(B)]
        for q in range(NQ - NG, NQ):
            for h in ss[q]:
                h.wait()

    return emb_kernel


_emb = _build()


def kernel(x, token_emb, pos_emb):
    return _emb(x.astype(jnp.int32), token_emb, pos_emb)
